# baseline (speedup 1.0000x reference)
import functools

import jax
import jax.numpy as jnp
from jax import lax
from jax.experimental import pallas as pl
from jax.experimental.pallas import tpu as pltpu
from jax.experimental.pallas import tpu_sc as plsc

N_ITEMS = 100000
N_USERS = 100000
K = 200
P = 50
B = 4096
L = 16
W = 128

NC = 2
NS = 16
NW = NC * NS
QW = B // NW
CH = 32
NCH = QW // CH
TBL = N_ITEMS + L

K_GRPS = [(0, 16 * g) for g in range(8)] + \
         [(1, 16 * g - W) for g in range(8, 12)] + [(1, K - L - W)]
K_TAIL_FRESH = 8
P_OFFS = [0, 16, 32, P - L]
P_TAIL_FRESH = 14


def _knn_body(iidx_hbm, u_hbm, nidxp_hbm, nsimp_hbm, profp_hbm, out_hbm,
              tbl, sbuf, iidx_v, u_v, nidx_v, nsim_v, prof_v, score_v, sem):
    wid = lax.axis_index("s") * NC + lax.axis_index("c")
    base = wid * QW
    lane = lax.iota(jnp.int32, L)
    zeros = jnp.zeros((L,), jnp.float32)

    def zero_body(j, carry):
        tbl[pl.ds(j * L, L)] = zeros
        return carry
    lax.fori_loop(0, TBL // L, zero_body, 0)

    sbuf[pl.ds(0, L)] = jnp.full((L,), -1, jnp.int32)
    sbuf[pl.ds(2 * L, L)] = jnp.full((L,), -2, jnp.int32)

    def histo_add(pidx):
        sk = jnp.sort(pidx)
        sbuf[pl.ds(L, L)] = sk
        prev = sbuf[pl.ds(L - 1, L)]
        nxt = sbuf[pl.ds(L + 1, L)]
        isfirst = sk != prev
        islast = sk != nxt
        run_start = plsc.cummax(jnp.where(isfirst, lane, 0))
        addcnt = (lane - run_start + 1).astype(jnp.float32)
        prevcnt = plsc.load_gather(tbl, [sk])
        plsc.store_scatter(tbl, [sk], prevcnt + addcnt, mask=islast)

    def chunk_body(c, carry):
        off = base + c * CH
        pltpu.sync_copy(iidx_hbm.at[pl.ds(2 * off, 2 * CH)], iidx_v)
        pltpu.sync_copy(u_hbm.at[pl.ds(off, CH)], u_v)
        cp1 = pltpu.async_copy(nidxp_hbm.at[iidx_v], nidx_v, sem)
        cp2 = pltpu.async_copy(nsimp_hbm.at[iidx_v], nsim_v, sem)
        cp3 = pltpu.async_copy(profp_hbm.at[u_v], prof_v, sem)
        cp1.wait()
        cp2.wait()
        cp3.wait()

        def qgrp_body(g, carry2):
            def q_body(qi, sv):
                q = g * L + qi
                for gi, poff in enumerate(P_OFFS):
                    pidx = prof_v[q, pl.ds(poff, L)]
                    if gi == len(P_OFFS) - 1:
                        pidx = jnp.where(lane >= P_TAIL_FRESH, pidx, N_ITEMS)
                    histo_add(pidx)
                acc = zeros
                for gi, (kr, kc) in enumerate(K_GRPS):
                    kidx = nidx_v[2 * q + kr, pl.ds(kc, L)]
                    ksim = nsim_v[2 * q + kr, pl.ds(kc, L)]
                    if gi == len(K_GRPS) - 1:
                        ksim = jnp.where(lane >= K_TAIL_FRESH, ksim, zeros)
                    cnt = plsc.load_gather(tbl, [kidx])
                    acc = acc + ksim * cnt
                for gi, poff in enumerate(P_OFFS):
                    pidx = prof_v[q, pl.ds(poff, L)]
                    if gi == len(P_OFFS) - 1:
                        pidx = jnp.where(lane >= P_TAIL_FRESH, pidx, N_ITEMS)
                    plsc.store_scatter(tbl, [pidx], zeros)
                return jnp.where(lane == qi, jnp.sum(acc), sv)
            sv = lax.fori_loop(0, L, q_body, zeros)
            score_v[pl.ds(c * CH + g * L, L)] = sv
            return carry2
        lax.fori_loop(0, CH // L, qgrp_body, 0)
        return carry
    lax.fori_loop(0, NCH, chunk_body, 0)

    pltpu.sync_copy(score_v, out_hbm.at[pl.ds(base, QW)])


_knn = functools.partial(
    pl.kernel,
    out_type=jax.ShapeDtypeStruct((B,), jnp.float32),
    mesh=plsc.VectorSubcoreMesh(core_axis_name="c", subcore_axis_name="s"),
    compiler_params=pltpu.CompilerParams(
        needs_layout_passes=False, use_tc_tiling_on_sc=True),
    scratch_types=[
        pltpu.VMEM((TBL,), jnp.float32),
        pltpu.VMEM((3 * L,), jnp.int32),
        pltpu.VMEM((2 * CH,), jnp.int32),
        pltpu.VMEM((CH,), jnp.int32),
        pltpu.VMEM((2 * CH, W), jnp.int32),
        pltpu.VMEM((2 * CH, W), jnp.float32),
        pltpu.VMEM((CH, W), jnp.int32),
        pltpu.VMEM((QW,), jnp.float32),
        pltpu.SemaphoreType.DMA,
    ],
)(_knn_body)


def kernel(u, i, nbr_idx, nbr_sim, user_prof):
    i = i.astype(jnp.int32)
    u = u.astype(jnp.int32)
    iidx = (2 * i[:, None] + jnp.arange(2, dtype=jnp.int32)).reshape(-1)
    nidxp = jnp.pad(nbr_idx, ((0, 0), (0, 2 * W - K))).reshape(-1, W)
    nsimp = jnp.pad(nbr_sim, ((0, 0), (0, 2 * W - K))).reshape(-1, W)
    profp = jnp.pad(user_prof, ((0, 0), (0, W - P)))
    return _knn(iidx, u, nidxp, nsimp, profp)


# speedup vs baseline: 2.6215x; 2.6215x over previous
import functools

import jax
import jax.numpy as jnp
from jax import lax
from jax.experimental import pallas as pl
from jax.experimental.pallas import tpu as pltpu
from jax.experimental.pallas import tpu_sc as plsc

N_ITEMS = 100000
N_USERS = 100000
K = 200
P = 50
B = 4096
L = 16
W = 128

NC = 2
NS = 16
NW = NC * NS
QW = B // NW
CH = 32
NCH = QW // CH
TBL = N_ITEMS + L

K_GRPS = [(0, 16 * g) for g in range(8)] + \
         [(1, 16 * g - W) for g in range(8, 12)] + [(1, K - L - W)]
K_TAIL_FRESH = 8
P_OFFS = [0, 16, 32, P - L]
P_TAIL_FRESH = 14


def _knn_body(i_hbm, u_hbm, nidxa_hbm, nidxb_hbm, nsima_hbm, nsimb_hbm,
              profp_hbm, out_hbm,
              tbl, sbuf, i_v, u_v, nidxa_v, nidxb_v, nsima_v, nsimb_v,
              prof_v, score_v, sem):
    wid = lax.axis_index("s") * NC + lax.axis_index("c")
    base = wid * QW
    lane = lax.iota(jnp.int32, L)
    zeros = jnp.zeros((L,), jnp.float32)

    def zero_body(j, carry):
        tbl[pl.ds(j * L, L)] = zeros
        return carry
    lax.fori_loop(0, TBL // L, zero_body, 0)

    sbuf[pl.ds(0, L)] = jnp.full((L,), -1, jnp.int32)
    sbuf[pl.ds(2 * L, L)] = jnp.full((L,), -2, jnp.int32)

    def histo_add(pidx):
        sk = jnp.sort(pidx)
        sbuf[pl.ds(L, L)] = sk
        prev = sbuf[pl.ds(L - 1, L)]
        nxt = sbuf[pl.ds(L + 1, L)]
        isfirst = sk != prev
        islast = sk != nxt
        run_start = plsc.cummax(jnp.where(isfirst, lane, 0))
        addcnt = (lane - run_start + 1).astype(jnp.float32)
        prevcnt = plsc.load_gather(tbl, [sk])
        plsc.store_scatter(tbl, [sk], prevcnt + addcnt, mask=islast)

    def chunk_body(c, carry):
        off = base + c * CH
        pltpu.sync_copy(i_hbm.at[pl.ds(off, CH)], i_v)
        pltpu.sync_copy(u_hbm.at[pl.ds(off, CH)], u_v)
        cps = [
            pltpu.async_copy(nidxa_hbm.at[i_v], nidxa_v, sem),
            pltpu.async_copy(nidxb_hbm.at[i_v], nidxb_v, sem),
            pltpu.async_copy(nsima_hbm.at[i_v], nsima_v, sem),
            pltpu.async_copy(nsimb_hbm.at[i_v], nsimb_v, sem),
            pltpu.async_copy(profp_hbm.at[u_v], prof_v, sem),
        ]
        for cp in cps:
            cp.wait()

        def qgrp_body(g, carry2):
            def q_body(qi, sv):
                q = g * L + qi
                for gi, poff in enumerate(P_OFFS):
                    pidx = prof_v[q, pl.ds(poff, L)]
                    if gi == len(P_OFFS) - 1:
                        pidx = jnp.where(lane >= P_TAIL_FRESH, pidx, N_ITEMS)
                    histo_add(pidx)
                acc = zeros
                for gi, (kr, kc) in enumerate(K_GRPS):
                    kidx = (nidxa_v if kr == 0 else nidxb_v)[q, pl.ds(kc, L)]
                    ksim = (nsima_v if kr == 0 else nsimb_v)[q, pl.ds(kc, L)]
                    if gi == len(K_GRPS) - 1:
                        ksim = jnp.where(lane >= K_TAIL_FRESH, ksim, zeros)
                    cnt = plsc.load_gather(tbl, [kidx])
                    acc = acc + ksim * cnt
                for gi, poff in enumerate(P_OFFS):
                    pidx = prof_v[q, pl.ds(poff, L)]
                    if gi == len(P_OFFS) - 1:
                        pidx = jnp.where(lane >= P_TAIL_FRESH, pidx, N_ITEMS)
                    plsc.store_scatter(tbl, [pidx], zeros)
                return jnp.where(lane == qi, jnp.sum(acc), sv)
            sv = lax.fori_loop(0, L, q_body, zeros)
            score_v[pl.ds(c * CH + g * L, L)] = sv
            return carry2
        lax.fori_loop(0, CH // L, qgrp_body, 0)
        return carry
    lax.fori_loop(0, NCH, chunk_body, 0)

    pltpu.sync_copy(score_v, out_hbm.at[pl.ds(base, QW)])


_knn = functools.partial(
    pl.kernel,
    out_type=jax.ShapeDtypeStruct((B,), jnp.float32),
    mesh=plsc.VectorSubcoreMesh(core_axis_name="c", subcore_axis_name="s"),
    compiler_params=pltpu.CompilerParams(
        needs_layout_passes=False, use_tc_tiling_on_sc=True),
    scratch_types=[
        pltpu.VMEM((TBL,), jnp.float32),
        pltpu.VMEM((3 * L,), jnp.int32),
        pltpu.VMEM((CH,), jnp.int32),
        pltpu.VMEM((CH,), jnp.int32),
        pltpu.VMEM((CH, W), jnp.int32),
        pltpu.VMEM((CH, W), jnp.int32),
        pltpu.VMEM((CH, W), jnp.float32),
        pltpu.VMEM((CH, W), jnp.float32),
        pltpu.VMEM((CH, W), jnp.int32),
        pltpu.VMEM((QW,), jnp.float32),
        pltpu.SemaphoreType.DMA,
    ],
)(_knn_body)


def kernel(u, i, nbr_idx, nbr_sim, user_prof):
    i = i.astype(jnp.int32)
    u = u.astype(jnp.int32)
    nidxa = nbr_idx[:, :W]
    nidxb = jnp.pad(nbr_idx[:, W:], ((0, 0), (0, 2 * W - K)))
    nsima = nbr_sim[:, :W]
    nsimb = jnp.pad(nbr_sim[:, W:], ((0, 0), (0, 2 * W - K)))
    profp = jnp.pad(user_prof, ((0, 0), (0, W - P)))
    return _knn(i, u, nidxa, nidxb, nsima, nsimb, profp)
